# Initial kernel scaffold; baseline (speedup 1.0000x reference)
import jax
import jax.numpy as jnp
from jax.experimental import pallas as pl


def kernel(x, edge_index, W1, b1, W2, b2, Wfc, bfc):
    raise NotImplementedError("write your pallas kernel here")


# speedup vs baseline: 17.0375x; 17.0375x over previous
import functools

import jax
import jax.numpy as jnp
from jax import lax
from jax.experimental import pallas as pl
from jax.experimental.pallas import tpu as pltpu
from jax.experimental.pallas import tpu_sc as plsc

N = 10000
D = 128
E = 320000

NC = 2
NS = 16
NT = NC * NS

CH = 128
E_FULL = E + N
CHUNKS = -(-E_FULL // (NT * CH))
E_PAD = CHUNKS * NT * CH
N_ACC = 10240
ROWS_PER_TILE = N_ACC // NS // CH

_MESH = dict(core_axis_name="c", subcore_axis_name="s", num_cores=NC,
             num_subcores=NS)


def _sc_degree(dst_t):
  mesh = plsc.VectorSubcoreMesh(**_MESH)
  per_tile = N_ACC // NS

  @functools.partial(
      pl.kernel,
      out_type=jax.ShapeDtypeStruct((NC, N_ACC), jnp.float32),
      mesh=mesh,
      scratch_types=[
          pltpu.VMEM((CHUNKS, CH), jnp.int32),
          pltpu.VMEM((CH,), jnp.float32),
          pltpu.VMEM((per_tile,), jnp.float32),
          pltpu.VMEM_SHARED((N_ACC,), jnp.float32),
      ],
  )
  def k(dst_hbm, out_hbm, idx_v, ones_v, stage_v, acc):
    cid = lax.axis_index("c")
    sid = lax.axis_index("s")
    wid = cid * NS + sid

    def zero_body(i, _):
      stage_v[pl.ds(i * 16, 16)] = jnp.zeros((16,), jnp.float32)
      return 0
    lax.fori_loop(0, per_tile // 16, zero_body, 0)
    for i in range(CH // 16):
      ones_v[pl.ds(i * 16, 16)] = jnp.ones((16,), jnp.float32)

    pltpu.sync_copy(stage_v, acc.at[pl.ds(sid * per_tile, per_tile)])
    pltpu.sync_copy(dst_hbm.at[wid], idx_v)
    plsc.subcore_barrier()

    def body(j, _):
      pltpu.sync_copy(ones_v, acc.at[idx_v.at[j]], add=True)
      return 0
    lax.fori_loop(0, CHUNKS, body, 0)

    plsc.subcore_barrier()
    pltpu.sync_copy(acc.at[pl.ds(sid * per_tile, per_tile)],
                    out_hbm.at[cid, pl.ds(sid * per_tile, per_tile)])

  return k(dst_t)


def _sc_scatter(y, src_t, dst_t):
  mesh = plsc.VectorSubcoreMesh(**_MESH)
  y_rows_per_tile = (N // NS) & ~7
  y_tail = N - NS * y_rows_per_tile

  @functools.partial(
      pl.kernel,
      out_type=jax.ShapeDtypeStruct((NC, N_ACC, D), jnp.float32),
      mesh=mesh,
      scratch_types=[
          pltpu.VMEM((CHUNKS, CH), jnp.int32),
          pltpu.VMEM((CHUNKS, CH), jnp.int32),
          pltpu.VMEM((CH, D), jnp.float32),
          pltpu.VMEM_SHARED((N_ACC, D), jnp.float32),
          pltpu.SemaphoreType.DMA,
      ],
  )
  def k(y_hbm, src_hbm, dst_hbm, out_hbm, srcv, dstv, rows, acc, sem):
    cid = lax.axis_index("c")
    sid = lax.axis_index("s")
    wid = cid * NS + sid

    @pl.when(cid == 0)
    def _():
      pltpu.sync_copy(y_hbm.at[pl.ds(sid * y_rows_per_tile, y_rows_per_tile)],
                      acc.at[pl.ds(sid * y_rows_per_tile, y_rows_per_tile)])
      @pl.when(sid == 0)
      def _():
        pltpu.sync_copy(y_hbm.at[pl.ds(NS * y_rows_per_tile, y_tail)],
                        acc.at[pl.ds(NS * y_rows_per_tile, y_tail)])

    @pl.when(cid == 1)
    def _():
      def zrow(r, _):
        for q in range(D // 16):
          rows[r, pl.ds(q * 16, 16)] = jnp.zeros((16,), jnp.float32)
        return 0
      lax.fori_loop(0, CH, zrow, 0)
      for i in range(ROWS_PER_TILE):
        pltpu.sync_copy(rows, acc.at[pl.ds((sid * ROWS_PER_TILE + i) * CH, CH)])

    pltpu.sync_copy(src_hbm.at[wid], srcv)
    pltpu.sync_copy(dst_hbm.at[wid], dstv)
    plsc.subcore_barrier()

    def body(j, _):
      pltpu.async_copy(y_hbm.at[srcv.at[j]], rows, sem).wait()
      pltpu.sync_copy(rows, acc.at[dstv.at[j]], add=True)
      return 0
    lax.fori_loop(0, CHUNKS, body, 0)

    plsc.subcore_barrier()
    for i in range(ROWS_PER_TILE):
      b = (sid * ROWS_PER_TILE + i) * CH
      pltpu.sync_copy(acc.at[pl.ds(b, CH)], out_hbm.at[cid].at[pl.ds(b, CH)])

  return k(y, src_t, dst_t)


BR = 1000


def _dinv(deg_blk):
  return jnp.where(deg_blk > 0, lax.rsqrt(deg_blk), 0.0)


def _tc_first(x, W1, degcol):
  def body(x_ref, w_ref, d_ref, o_ref):
    d = _dinv(d_ref[...])
    o_ref[...] = d * jnp.dot(x_ref[...], w_ref[...],
                             preferred_element_type=jnp.float32)
  return pl.pallas_call(
      body,
      grid=(N // BR,),
      in_specs=[
          pl.BlockSpec((BR, D), lambda i: (i, 0)),
          pl.BlockSpec((D, D), lambda i: (0, 0)),
          pl.BlockSpec((BR, 1), lambda i: (i, 0)),
      ],
      out_specs=pl.BlockSpec((BR, D), lambda i: (i, 0)),
      out_shape=jax.ShapeDtypeStruct((N, D), jnp.float32),
  )(x, W1, degcol)


def _tc_mid(P0, P1, degcol, b1, W2):
  def body(p0_ref, p1_ref, d_ref, b_ref, w_ref, o_ref):
    d = _dinv(d_ref[...])
    h = jnp.maximum(d * (p0_ref[...] + p1_ref[...]) + b_ref[...], 0.0)
    o_ref[...] = d * jnp.dot(h, w_ref[...], preferred_element_type=jnp.float32)
  return pl.pallas_call(
      body,
      grid=(N // BR,),
      in_specs=[
          pl.BlockSpec((BR, D), lambda i: (i, 0)),
          pl.BlockSpec((BR, D), lambda i: (i, 0)),
          pl.BlockSpec((BR, 1), lambda i: (i, 0)),
          pl.BlockSpec((1, D), lambda i: (0, 0)),
          pl.BlockSpec((D, D), lambda i: (0, 0)),
      ],
      out_specs=pl.BlockSpec((BR, D), lambda i: (i, 0)),
      out_shape=jax.ShapeDtypeStruct((N, D), jnp.float32),
  )(P0, P1, degcol, b1.reshape(1, D), W2)


def _tc_final(Q0, Q1, degcol, b2, Wfc, bfc):
  dout = Wfc.shape[1]

  def body(q0_ref, q1_ref, d_ref, b_ref, w_ref, bf_ref, o_ref):
    d = _dinv(d_ref[...])
    h = jnp.maximum(d * (q0_ref[...] + q1_ref[...]) + b_ref[...], 0.0)
    logits = jnp.dot(h, w_ref[...], preferred_element_type=jnp.float32)
    logits = logits + bf_ref[...]
    m = jnp.max(logits, axis=1, keepdims=True)
    lse = m + jnp.log(jnp.sum(jnp.exp(logits - m), axis=1, keepdims=True))
    o_ref[...] = logits - lse

  return pl.pallas_call(
      body,
      grid=(N // BR,),
      in_specs=[
          pl.BlockSpec((BR, D), lambda i: (i, 0)),
          pl.BlockSpec((BR, D), lambda i: (i, 0)),
          pl.BlockSpec((BR, 1), lambda i: (i, 0)),
          pl.BlockSpec((1, D), lambda i: (0, 0)),
          pl.BlockSpec((D, dout), lambda i: (0, 0)),
          pl.BlockSpec((1, dout), lambda i: (0, 0)),
      ],
      out_specs=pl.BlockSpec((BR, dout), lambda i: (i, 0)),
      out_shape=jax.ShapeDtypeStruct((N, dout), jnp.float32),
  )(Q0, Q1, degcol, b2.reshape(1, D), Wfc, bfc.reshape(1, dout))


def kernel(x, edge_index, W1, b1, W2, b2, Wfc, bfc):
  loop = jnp.arange(N, dtype=jnp.int32)
  src = jnp.concatenate([edge_index[0].astype(jnp.int32), loop])
  dst = jnp.concatenate([edge_index[1].astype(jnp.int32), loop])
  npad = E_PAD - E_FULL
  src = jnp.concatenate([src, jnp.zeros((npad,), jnp.int32)])
  dst = jnp.concatenate([dst, jnp.full((npad,), N, jnp.int32)])
  src_t = src.reshape(NT, CHUNKS, CH)
  dst_t = dst.reshape(NT, CHUNKS, CH)

  degP = _sc_degree(dst_t)
  degcol = (degP[0] + degP[1])[:N].reshape(N, 1)

  y1 = _tc_first(x, W1, degcol)
  P = _sc_scatter(y1, src_t, dst_t)
  y2 = _tc_mid(P[0, :N], P[1, :N], degcol, b1, W2)
  Q = _sc_scatter(y2, src_t, dst_t)
  return _tc_final(Q[0, :N], Q[1, :N], degcol, b2, Wfc, bfc)
